# Initial kernel scaffold; baseline (speedup 1.0000x reference)
import jax
import jax.numpy as jnp
from jax.experimental import pallas as pl


def kernel(feat, w, b):
    raise NotImplementedError("write your pallas kernel here")


# speedup vs baseline: 1.4212x; 1.4212x over previous
import jax
import jax.numpy as jnp
from jax.experimental import pallas as pl
from jax.experimental.pallas import tpu as pltpu


def _fused_kernel(x_ref, w_ref, b_ref, o_ref):
    x = jnp.maximum(x_ref[...], 0.0)
    acc = jnp.dot(x, w_ref[...], preferred_element_type=jnp.float32)
    o_ref[...] = (acc + b_ref[...]).astype(o_ref.dtype)


@jax.jit
def kernel(feat, w, b):
    B, D = feat.shape
    _, N = w.shape

    tb = min(1024, max(8, (B + 7) // 8 * 8))
    b_pad = (B + tb - 1) // tb * tb
    feat_p = jnp.pad(feat, ((0, b_pad - B), (0, 0))) if b_pad != B else feat

    out = pl.pallas_call(
        _fused_kernel,
        out_shape=jax.ShapeDtypeStruct((b_pad, N), feat.dtype),
        grid=(b_pad // tb,),
        in_specs=[
            pl.BlockSpec((tb, D), lambda i: (i, 0)),
            pl.BlockSpec((D, N), lambda i: (0, 0)),
            pl.BlockSpec((1, N), lambda i: (0, 0)),
        ],
        out_specs=pl.BlockSpec((tb, N), lambda i: (i, 0)),
        compiler_params=pltpu.CompilerParams(
            dimension_semantics=("parallel",),
        ),
    )(feat_p, w, b.reshape(1, N))

    return out[:B] if b_pad != B else out


# speedup vs baseline: 1.5967x; 1.1235x over previous
import jax
import jax.numpy as jnp
from jax.experimental import pallas as pl
from jax.experimental.pallas import tpu as pltpu


def _fused_kernel(x_ref, w_ref, b_ref, o_ref):
    x = jnp.maximum(x_ref[...], 0.0)
    acc = jnp.dot(x, w_ref[...], preferred_element_type=jnp.float32)
    o_ref[...] = (acc + b_ref[...]).astype(o_ref.dtype)


@jax.jit
def kernel(feat, w, b):
    B, D = feat.shape
    _, N = w.shape

    tb = min(4096, max(8, (B + 7) // 8 * 8))
    b_pad = (B + tb - 1) // tb * tb
    feat_p = jnp.pad(feat, ((0, b_pad - B), (0, 0))) if b_pad != B else feat

    out = pl.pallas_call(
        _fused_kernel,
        out_shape=jax.ShapeDtypeStruct((b_pad, N), feat.dtype),
        grid=(b_pad // tb,),
        in_specs=[
            pl.BlockSpec((tb, D), lambda i: (i, 0)),
            pl.BlockSpec((D, N), lambda i: (0, 0)),
            pl.BlockSpec((1, N), lambda i: (0, 0)),
        ],
        out_specs=pl.BlockSpec((tb, N), lambda i: (i, 0)),
        compiler_params=pltpu.CompilerParams(
            dimension_semantics=("parallel",),
        ),
    )(feat_p, w, b.reshape(1, N))

    return out[:B] if b_pad != B else out
